# baseline (speedup 1.0000x reference)
import jax
import jax.numpy as jnp
from jax import lax
from jax.experimental import pallas as pl
from jax.experimental.pallas import tpu as pltpu
from jax.experimental.pallas import tpu_sc as plsc

N = 10000
E = 320000
D = 128
L = 16
NC = 2
NS = 16
CH = 128
P = 8
NCH0 = 104
NCH1 = 56
TOTCH = NS * (NCH0 + NCH1)
EPAD = TOTCH * CH
SC1_BASE = NS * NCH0
ROWS_PER_SUB = 624
TAIL = N - NS * ROWS_PER_SUB
ZR = 16


def _sc_body(x_hbm, src_hbm, dst_hbm, vals_hbm, part_hbm,
             acc, src_v, dst_v, vals_v, rows, zbuf):
  cid = lax.axis_index("c")
  sid = lax.axis_index("s")
  nph = jnp.where(cid == 0, NCH0 // P, NCH1 // P)
  cb = jnp.where(cid == 0, sid * NCH0, SC1_BASE + sid * NCH1)

  def zrow(r, carry):
    for k in range(D // L):
      zbuf[r, pl.ds(k * L, L)] = jnp.zeros((L,), jnp.float32)
    return carry
  lax.fori_loop(0, ZR, zrow, 0)
  base = sid * ROWS_PER_SUB

  def zcopy(i, carry):
    pltpu.sync_copy(zbuf, acc.at[pl.ds(base + i * ZR, ZR)])
    return carry
  lax.fori_loop(0, ROWS_PER_SUB // ZR, zcopy, 0)

  @pl.when(sid == NS - 1)
  def _zero_tail():
    pltpu.sync_copy(zbuf.at[pl.ds(0, TAIL)], acc.at[pl.ds(NS * ROWS_PER_SUB, TAIL)])
  plsc.subcore_barrier()

  dnums = lax.GatherDimensionNumbers(
      offset_dims=(), collapsed_slice_dims=(0,), start_index_map=(0,))

  def phase(ph, carry):
    start = pl.multiple_of(cb + ph * P, 8)
    pltpu.sync_copy(src_hbm.at[pl.ds(start, P)], src_v)
    pltpu.sync_copy(dst_hbm.at[pl.ds(start, P)], dst_v)
    pltpu.sync_copy(vals_hbm.at[pl.ds(start, P)], vals_v)

    def chunk(r, ccarry):
      pltpu.sync_copy(x_hbm.at[src_v.at[r]], rows)

      def group(g, gcarry):
        vv = vals_v[r, pl.ds(g * L, L)]
        for j in range(L):
          e = g * L + j
          vj = lax.gather(vv, jnp.full((L, 1), j, jnp.int32), dnums,
                          slice_sizes=(1,),
                          mode=lax.GatherScatterMode.PROMISE_IN_BOUNDS)
          for k in range(D // L):
            sl = pl.ds(k * L, L)
            rows[e, sl] = rows[e, sl] * vj
        return gcarry
      lax.fori_loop(0, CH // L, group, 0)

      pltpu.sync_copy(rows, acc.at[dst_v.at[r]], add=True)
      return ccarry
    lax.fori_loop(0, P, chunk, 0)
    return carry
  lax.fori_loop(0, nph, phase, 0)

  plsc.subcore_barrier()
  pltpu.sync_copy(acc.at[pl.ds(base, ROWS_PER_SUB)],
                  part_hbm.at[cid, pl.ds(base, ROWS_PER_SUB)])

  @pl.when(sid == NS - 1)
  def _write_tail():
    pltpu.sync_copy(acc.at[pl.ds(NS * ROWS_PER_SUB, TAIL)],
                    part_hbm.at[cid, pl.ds(NS * ROWS_PER_SUB, TAIL)])


@jax.jit
def _sc_spmm(x, src_p, dst_p, vals_p):
  mesh = plsc.VectorSubcoreMesh(core_axis_name="c", subcore_axis_name="s")
  return pl.kernel(
      _sc_body,
      out_type=jax.ShapeDtypeStruct((NC, N, D), jnp.float32),
      mesh=mesh,
      scratch_types=[
          pltpu.VMEM_SHARED((N, D), jnp.float32),
          pltpu.VMEM((P, CH), jnp.int32),
          pltpu.VMEM((P, CH), jnp.int32),
          pltpu.VMEM((P, CH), jnp.float32),
          pltpu.VMEM((CH, D), jnp.float32),
          pltpu.VMEM((ZR, D), jnp.float32),
      ],
  )(x, src_p, dst_p, vals_p)


def _add_body(p_ref, o_ref):
  o_ref[...] = p_ref[0] + p_ref[1]


@jax.jit
def _combine(partials):
  rb = 1000
  return pl.pallas_call(
      _add_body,
      grid=(N // rb,),
      in_specs=[pl.BlockSpec((NC, rb, D), lambda i: (0, i, 0))],
      out_specs=pl.BlockSpec((rb, D), lambda i: (i, 0)),
      out_shape=jax.ShapeDtypeStruct((N, D), jnp.float32),
  )(partials)


def kernel(t, x, edge_index, edge_vals):
  src = edge_index[0].astype(jnp.int32)
  dst = edge_index[1].astype(jnp.int32)
  vals = edge_vals.astype(jnp.float32)
  pad = EPAD - E
  src_p = jnp.pad(src, (0, pad)).reshape(TOTCH, CH)
  dst_p = jnp.pad(dst, (0, pad)).reshape(TOTCH, CH)
  vals_p = jnp.pad(vals, (0, pad)).reshape(TOTCH, CH)
  partials = _sc_spmm(x, src_p, dst_p, vals_p)
  return _combine(partials)


# speedup vs baseline: 1.0108x; 1.0108x over previous
import jax
import jax.numpy as jnp
from jax import lax
from jax.experimental import pallas as pl
from jax.experimental.pallas import tpu as pltpu
from jax.experimental.pallas import tpu_sc as plsc

N = 10000
E = 320000
D = 128
L = 16
NC = 2
NS = 16
CH = 128
P = 8
NCH0 = 104
NCH1 = 56
TOTCH = NS * (NCH0 + NCH1)
EPAD = TOTCH * CH
SC1_BASE = NS * NCH0
ROWS_PER_SUB = 624
TAIL = N - NS * ROWS_PER_SUB
ZR = 16


def _sc_body(x_hbm, src_hbm, dst_hbm, vals_hbm, part_hbm,
             acc, src_s, dst_s, vals_s, src_f, dst_f, vals_f, rows, zbuf):
  cid = lax.axis_index("c")
  sid = lax.axis_index("s")

  def zrow(r, carry):
    for k in range(D // L):
      zbuf[r, pl.ds(k * L, L)] = jnp.zeros((L,), jnp.float32)
    return carry
  lax.fori_loop(0, ZR, zrow, 0)
  base = sid * ROWS_PER_SUB

  def zcopy(i, carry):
    pltpu.sync_copy(zbuf, acc.at[pl.ds(base + i * ZR, ZR)])
    return carry
  lax.fori_loop(0, ROWS_PER_SUB // ZR, zcopy, 0)

  @pl.when(sid == NS - 1)
  def _zero_tail():
    pltpu.sync_copy(zbuf.at[pl.ds(0, TAIL)], acc.at[pl.ds(NS * ROWS_PER_SUB, TAIL)])
  plsc.subcore_barrier()

  dnums = lax.GatherDimensionNumbers(
      offset_dims=(), collapsed_slice_dims=(0,), start_index_map=(0,))

  def scale_rows(vals_ref, r):
    def group(g, gcarry):
      vv = vals_ref[r, pl.ds(g * L, L)]
      for j in range(L):
        e = g * L + j
        vj = lax.gather(vv, jnp.full((L, 1), j, jnp.int32), dnums,
                        slice_sizes=(1,),
                        mode=lax.GatherScatterMode.PROMISE_IN_BOUNDS)
        for k in range(D // L):
          sl = pl.ds(k * L, L)
          rows[e, sl] = rows[e, sl] * vj
      return gcarry
    lax.fori_loop(0, CH // L, group, 0)

  def do_chunk(src_ref, dst_ref, vals_ref, r):
    pltpu.sync_copy(x_hbm.at[src_ref.at[r]], rows)
    scale_rows(vals_ref, r)
    pltpu.sync_copy(rows, acc.at[dst_ref.at[r]], add=True)

  @pl.when(cid == 0)
  def _sc0_loop():
    cb = sid * NCH0

    def phase(ph, carry):
      start = pl.multiple_of(cb + ph * P, 8)
      pltpu.sync_copy(src_hbm.at[pl.ds(start, P)], src_s)
      pltpu.sync_copy(dst_hbm.at[pl.ds(start, P)], dst_s)
      pltpu.sync_copy(vals_hbm.at[pl.ds(start, P)], vals_s)

      def chunk(r, ccarry):
        do_chunk(src_s, dst_s, vals_s, r)
        return ccarry
      lax.fori_loop(0, P, chunk, 0)
      return carry
    lax.fori_loop(0, NCH0 // P, phase, 0)

  @pl.when(cid == 1)
  def _sc1_loop():
    cb = pl.multiple_of(SC1_BASE + sid * NCH1, 8)
    pltpu.sync_copy(src_hbm.at[pl.ds(cb, NCH1)], src_f)
    pltpu.sync_copy(dst_hbm.at[pl.ds(cb, NCH1)], dst_f)
    pltpu.sync_copy(vals_hbm.at[pl.ds(cb, NCH1)], vals_f)

    def chunk(r, ccarry):
      do_chunk(src_f, dst_f, vals_f, r)
      return ccarry
    lax.fori_loop(0, NCH1, chunk, 0)

  plsc.subcore_barrier()
  pltpu.sync_copy(acc.at[pl.ds(base, ROWS_PER_SUB)],
                  part_hbm.at[cid, pl.ds(base, ROWS_PER_SUB)])

  @pl.when(sid == NS - 1)
  def _write_tail():
    pltpu.sync_copy(acc.at[pl.ds(NS * ROWS_PER_SUB, TAIL)],
                    part_hbm.at[cid, pl.ds(NS * ROWS_PER_SUB, TAIL)])


@jax.jit
def _sc_spmm(x, src_p, dst_p, vals_p):
  mesh = plsc.VectorSubcoreMesh(core_axis_name="c", subcore_axis_name="s")
  return pl.kernel(
      _sc_body,
      out_type=jax.ShapeDtypeStruct((NC, N, D), jnp.float32),
      mesh=mesh,
      scratch_types=[
          pltpu.VMEM_SHARED((N, D), jnp.float32),
          pltpu.VMEM((P, CH), jnp.int32),
          pltpu.VMEM((P, CH), jnp.int32),
          pltpu.VMEM((P, CH), jnp.float32),
          pltpu.VMEM((NCH1, CH), jnp.int32),
          pltpu.VMEM((NCH1, CH), jnp.int32),
          pltpu.VMEM((NCH1, CH), jnp.float32),
          pltpu.VMEM((CH, D), jnp.float32),
          pltpu.VMEM((ZR, D), jnp.float32),
      ],
  )(x, src_p, dst_p, vals_p)


def _add_body(p_ref, o_ref):
  o_ref[...] = p_ref[0] + p_ref[1]


@jax.jit
def _combine(partials):
  rb = 1000
  return pl.pallas_call(
      _add_body,
      grid=(N // rb,),
      in_specs=[pl.BlockSpec((NC, rb, D), lambda i: (0, i, 0))],
      out_specs=pl.BlockSpec((rb, D), lambda i: (i, 0)),
      out_shape=jax.ShapeDtypeStruct((N, D), jnp.float32),
  )(partials)


def kernel(t, x, edge_index, edge_vals):
  src = edge_index[0].astype(jnp.int32)
  dst = edge_index[1].astype(jnp.int32)
  vals = edge_vals.astype(jnp.float32)
  pad = EPAD - E
  src_p = jnp.pad(src, (0, pad)).reshape(TOTCH, CH)
  dst_p = jnp.pad(dst, (0, pad)).reshape(TOTCH, CH)
  vals_p = jnp.pad(vals, (0, pad)).reshape(TOTCH, CH)
  partials = _sc_spmm(x, src_p, dst_p, vals_p)
  return _combine(partials)


# speedup vs baseline: 1.4967x; 1.4808x over previous
import functools

import jax
import jax.numpy as jnp
from jax import lax
from jax.experimental import pallas as pl
from jax.experimental.pallas import tpu as pltpu
from jax.experimental.pallas import tpu_sc as plsc

N = 10000
E = 320000
D = 128
L = 16
NC = 2
NS = 16
NW = NC * NS
CH = 128
NCH = 79
EPW = NCH * CH
EPAD = NW * EPW
ROWS_PER_SUB = 624
TAIL = N - NS * ROWS_PER_SUB
ZR = 16


def _sc_body(x_hbm, src_hbm, dst_hbm, vals_hbm, part_hbm,
             acc, src_v, dst_v, vals_v, rows, zbuf):
  cid = lax.axis_index("c")
  sid = lax.axis_index("s")
  wid = cid * NS + sid

  pltpu.sync_copy(src_hbm.at[wid], src_v)
  pltpu.sync_copy(dst_hbm.at[wid], dst_v)
  pltpu.sync_copy(vals_hbm.at[wid], vals_v)

  def zrow(r, carry):
    for k in range(D // L):
      zbuf[r, pl.ds(k * L, L)] = jnp.zeros((L,), jnp.float32)
    return carry
  lax.fori_loop(0, ZR, zrow, 0)
  base = sid * ROWS_PER_SUB

  def zcopy(i, carry):
    pltpu.sync_copy(zbuf, acc.at[pl.ds(base + i * ZR, ZR)])
    return carry
  lax.fori_loop(0, ROWS_PER_SUB // ZR, zcopy, 0)

  @pl.when(sid == NS - 1)
  def _zero_tail():
    pltpu.sync_copy(zbuf.at[pl.ds(0, TAIL)], acc.at[pl.ds(NS * ROWS_PER_SUB, TAIL)])
  plsc.subcore_barrier()

  def chunk(c, carry):
    pltpu.sync_copy(x_hbm.at[src_v.at[c]], rows)
    def group(g, gcarry):
      vv = vals_v[c, pl.ds(g * L, L)]
      dnums = lax.GatherDimensionNumbers(
          offset_dims=(), collapsed_slice_dims=(0,), start_index_map=(0,))
      for j in range(L):
        e = g * L + j
        vj = lax.gather(vv, jnp.full((L, 1), j, jnp.int32), dnums,
                        slice_sizes=(1,),
                        mode=lax.GatherScatterMode.PROMISE_IN_BOUNDS)
        for k in range(D // L):
          sl = pl.ds(k * L, L)
          rows[e, sl] = rows[e, sl] * vj
      return gcarry
    lax.fori_loop(0, CH // L, group, 0)
    pltpu.sync_copy(rows, acc.at[dst_v.at[c]], add=True)
    return carry
  lax.fori_loop(0, NCH, chunk, 0)

  plsc.subcore_barrier()
  pltpu.sync_copy(acc.at[pl.ds(base, ROWS_PER_SUB)],
                  part_hbm.at[cid, pl.ds(base, ROWS_PER_SUB)])

  @pl.when(sid == NS - 1)
  def _write_tail():
    pltpu.sync_copy(acc.at[pl.ds(NS * ROWS_PER_SUB, TAIL)],
                    part_hbm.at[cid, pl.ds(NS * ROWS_PER_SUB, TAIL)])


@jax.jit
def _sc_spmm(x, src_p, dst_p, vals_p):
  mesh = plsc.VectorSubcoreMesh(core_axis_name="c", subcore_axis_name="s")
  return pl.kernel(
      _sc_body,
      out_type=jax.ShapeDtypeStruct((NC, N, D), jnp.float32),
      mesh=mesh,
      scratch_types=[
          pltpu.VMEM_SHARED((N, D), jnp.float32),
          pltpu.VMEM((NCH, CH), jnp.int32),
          pltpu.VMEM((NCH, CH), jnp.int32),
          pltpu.VMEM((NCH, CH), jnp.float32),
          pltpu.VMEM((CH, D), jnp.float32),
          pltpu.VMEM((ZR, D), jnp.float32),
      ],
  )(x, src_p, dst_p, vals_p)


def _add_body(p_ref, o_ref):
  o_ref[...] = p_ref[0] + p_ref[1]


@jax.jit
def _combine(partials):
  rb = 1000
  return pl.pallas_call(
      _add_body,
      grid=(N // rb,),
      in_specs=[pl.BlockSpec((NC, rb, D), lambda i: (0, i, 0))],
      out_specs=pl.BlockSpec((rb, D), lambda i: (i, 0)),
      out_shape=jax.ShapeDtypeStruct((N, D), jnp.float32),
  )(partials)


def kernel(t, x, edge_index, edge_vals):
  src = edge_index[0].astype(jnp.int32)
  dst = edge_index[1].astype(jnp.int32)
  vals = edge_vals.astype(jnp.float32)
  pad = EPAD - E
  src = jnp.pad(src, (0, pad)).reshape(NW, NCH, CH)
  dst = jnp.pad(dst, (0, pad)).reshape(NW, NCH, CH)
  vals = jnp.pad(vals, (0, pad)).reshape(NW, NCH, CH)
  xx = jnp.concatenate([x, x], axis=0)
  src = src + jnp.where(jnp.arange(NW)[:, None, None] >= NS, N, 0)
  partials = _sc_spmm(xx, src, dst, vals)
  return _combine(partials)
